# baseline (speedup 1.0000x reference)
import functools

import jax
import jax.numpy as jnp
from jax import lax
from jax.experimental import pallas as pl
from jax.experimental.pallas import tpu as pltpu
from jax.experimental.pallas import tpu_sc as plsc

VOCAB = 100000
EMB = 64
UNITS = 64
BATCH = 1024
SEQ = 50

NC = 2
NS = 16
NW = NC * NS
FPW = EMB // NW


def _fgather_body(tbl_hbm, idx_hbm, out_hbm, row_v, idxs_v, outs_v, sem):
    wid = lax.axis_index("s") * NC + lax.axis_index("c")
    z16 = jnp.zeros((16,), jnp.int32)
    for f in range(FPW):
        e = wid * FPW + f
        pltpu.sync_copy(tbl_hbm.at[pl.ds(e, 1)], row_v)

        def step(s, _):
            pltpu.sync_copy(idx_hbm.at[pl.ds(s * BATCH, BATCH)], idxs_v)

            def grp(k, __):
                idx16 = idxs_v[pl.ds(k * 16, 16)]
                vals = plsc.load_gather(row_v, [z16, idx16])
                outs_v[0, 0, pl.ds(k * 16, 16)] = vals
                return __

            lax.fori_loop(0, BATCH // 16, grp, 0)
            pltpu.sync_copy(outs_v, out_hbm.at[pl.ds(s, 1), pl.ds(e, 1), :])
            return _

        lax.fori_loop(0, SEQ, step, 0)


def _sc_fgather(tbl_t, idx):
    mesh = plsc.VectorSubcoreMesh(core_axis_name="c", subcore_axis_name="s")
    f = functools.partial(
        pl.kernel,
        mesh=mesh,
        out_type=jax.ShapeDtypeStruct((SEQ, EMB, BATCH), jnp.float32),
        scratch_types=[
            pltpu.VMEM((1, VOCAB), jnp.float32),
            pltpu.VMEM((BATCH,), jnp.int32),
            pltpu.VMEM((1, 1, BATCH), jnp.float32),
            pltpu.SemaphoreType.DMA,
        ],
        compiler_params=pltpu.CompilerParams(use_tc_tiling_on_sc=True,
                                             needs_layout_passes=False),
    )(_fgather_body)
    return f(tbl_t, idx)


SPI = 5


def _gru_body(emb_ref, w_ref, u_ref, bt_ref, seq_ref, last_ref, h_ref):
    g = pl.program_id(0)

    @pl.when(g == 0)
    def _():
        h_ref[...] = jnp.zeros((UNITS, BATCH), jnp.float32)

    W = w_ref[...]
    U = u_ref[...]
    bi = bt_ref[:, 0:1]
    br = bt_ref[:, 1:2]
    h = h_ref[...]

    for j in range(SPI):
        x_t = emb_ref[j]
        xp = lax.dot_general(W, x_t, (((0,), (0,)), ((), ())),
                             preferred_element_type=jnp.float32) + bi
        hp = lax.dot_general(U, h, (((0,), (0,)), ((), ())),
                             preferred_element_type=jnp.float32) + br
        z = jax.nn.sigmoid(xp[0:UNITS] + hp[0:UNITS])
        r = jax.nn.sigmoid(xp[UNITS:2 * UNITS] + hp[UNITS:2 * UNITS])
        hh = jnp.tanh(xp[2 * UNITS:] + r * hp[2 * UNITS:])
        h = z * h + (1.0 - z) * hh
        seq_ref[j] = h

    h_ref[...] = h

    @pl.when(g == SEQ // SPI - 1)
    def _():
        last_ref[...] = h


def _tc_gru(embt, W, U, bt):
    return pl.pallas_call(
        _gru_body,
        grid=(SEQ // SPI,),
        in_specs=[
            pl.BlockSpec((SPI, EMB, BATCH), lambda s: (s, 0, 0)),
            pl.BlockSpec((EMB, 3 * UNITS), lambda s: (0, 0)),
            pl.BlockSpec((UNITS, 3 * UNITS), lambda s: (0, 0)),
            pl.BlockSpec((3 * UNITS, 2), lambda s: (0, 0)),
        ],
        out_specs=[
            pl.BlockSpec((SPI, UNITS, BATCH), lambda s: (s, 0, 0)),
            pl.BlockSpec((UNITS, BATCH), lambda s: (0, 0)),
        ],
        out_shape=[
            jax.ShapeDtypeStruct((SEQ, UNITS, BATCH), jnp.float32),
            jax.ShapeDtypeStruct((UNITS, BATCH), jnp.float32),
        ],
        scratch_shapes=[pltpu.VMEM((UNITS, BATCH), jnp.float32)],
    )(embt, W, U, bt)


def kernel(x, initial, emb_table, W, U, b):
    del initial
    idx = jnp.swapaxes(x, 0, 1).reshape(-1).astype(jnp.int32)
    embt = _sc_fgather(emb_table.T, idx)
    seq_t, last_t = _tc_gru(embt, W, U, b.T)
    seq_out = jnp.transpose(seq_t, (2, 0, 1))
    last_state = jnp.transpose(last_t, (1, 0))
    embedded = jnp.transpose(embt, (2, 0, 1))
    return (seq_out, last_state, embedded)


# speedup vs baseline: 1.8456x; 1.8456x over previous
import functools

import jax
import jax.numpy as jnp
from jax import lax
from jax.experimental import pallas as pl
from jax.experimental.pallas import tpu as pltpu
from jax.experimental.pallas import tpu_sc as plsc

VOCAB = 100000
EMB = 64
UNITS = 64
BATCH = 1024
SEQ = 50

TBLK = 8192
TGRID = -(-VOCAB // TBLK)


def _tr_body(tin_ref, out_ref):
    v = tin_ref[...]
    out_ref[:, 0:EMB] = v.T


def _tc_prep(tbl_t):
    return pl.pallas_call(
        _tr_body,
        grid=(TGRID,),
        in_specs=[pl.BlockSpec((EMB, TBLK), lambda i: (0, i))],
        out_specs=pl.BlockSpec((TBLK, 2 * EMB), lambda i: (i, 0)),
        out_shape=jax.ShapeDtypeStruct((VOCAB, 2 * EMB), jnp.float32),
    )(tbl_t)


NC = 2
NS = 16
NW = NC * NS
TOTAL_ROWS = BATCH * SEQ
ROWS_PER_W = TOTAL_ROWS // NW
PASS_ROWS = 800
CHUNK = 128
_CHUNKS = []
_off = 0
while _off < PASS_ROWS:
    _c = min(CHUNK, PASS_ROWS - _off)
    _CHUNKS.append((_off, _c))
    _off += _c


def _gather_body(table_hbm, idx_hbm, out_hbm, idx_v, rows_v, sem):
    wid = lax.axis_index("s") * NC + lax.axis_index("c")
    base = wid * ROWS_PER_W
    pltpu.sync_copy(idx_hbm.at[pl.ds(base, ROWS_PER_W)], idx_v)
    for p in range(ROWS_PER_W // PASS_ROWS):
        pbase = p * PASS_ROWS
        copies = []
        for off, c in _CHUNKS:
            cp = pltpu.async_copy(
                table_hbm.at[idx_v.at[pl.ds(pbase + off, c)]],
                rows_v.at[pl.ds(off, c)],
                sem,
            )
            copies.append(cp)
        for cp in copies:
            cp.wait()
        pltpu.sync_copy(rows_v, out_hbm.at[pl.ds(base + pbase, PASS_ROWS)])


def _sc_gather(table128, idx):
    mesh = plsc.VectorSubcoreMesh(core_axis_name="c", subcore_axis_name="s")
    f = functools.partial(
        pl.kernel,
        mesh=mesh,
        out_type=jax.ShapeDtypeStruct((TOTAL_ROWS, 2 * EMB), jnp.float32),
        scratch_types=[
            pltpu.VMEM((ROWS_PER_W,), jnp.int32),
            pltpu.VMEM((PASS_ROWS, 2 * EMB), jnp.float32),
            pltpu.SemaphoreType.DMA,
        ],
        compiler_params=pltpu.CompilerParams(use_tc_tiling_on_sc=True),
    )(_gather_body)
    return f(table128, idx)


SPI = 10


def _gru_body(emb_ref, w_ref, u_ref, bt_ref, seq_ref, last_ref,
              embt_ref, h_ref):
    g = pl.program_id(0)

    @pl.when(g == 0)
    def _():
        h_ref[...] = jnp.zeros((UNITS, BATCH), jnp.float32)

    W = w_ref[...]
    U = u_ref[...]
    bi = bt_ref[:, 0:1]
    br = bt_ref[:, 1:2]
    h = h_ref[...]

    for j in range(SPI):
        buf = emb_ref[j]
        x_t = buf.T[0:EMB]
        xp = lax.dot_general(W, x_t, (((0,), (0,)), ((), ())),
                             preferred_element_type=jnp.float32) + bi
        hp = lax.dot_general(U, h, (((0,), (0,)), ((), ())),
                             preferred_element_type=jnp.float32) + br
        z = jax.nn.sigmoid(xp[0:UNITS] + hp[0:UNITS])
        r = jax.nn.sigmoid(xp[UNITS:2 * UNITS] + hp[UNITS:2 * UNITS])
        hh = jnp.tanh(xp[2 * UNITS:] + r * hp[2 * UNITS:])
        h = z * h + (1.0 - z) * hh
        seq_ref[j] = h
        embt_ref[j] = x_t

    h_ref[...] = h

    @pl.when(g == SEQ // SPI - 1)
    def _():
        last_ref[...] = h


def _tc_gru(emb128, W, U, bt):
    return pl.pallas_call(
        _gru_body,
        grid=(SEQ // SPI,),
        in_specs=[
            pl.BlockSpec((SPI, BATCH, 2 * EMB), lambda s: (s, 0, 0)),
            pl.BlockSpec((EMB, 3 * UNITS), lambda s: (0, 0)),
            pl.BlockSpec((UNITS, 3 * UNITS), lambda s: (0, 0)),
            pl.BlockSpec((3 * UNITS, 2), lambda s: (0, 0)),
        ],
        out_specs=[
            pl.BlockSpec((SPI, UNITS, BATCH), lambda s: (s, 0, 0)),
            pl.BlockSpec((UNITS, BATCH), lambda s: (0, 0)),
            pl.BlockSpec((SPI, EMB, BATCH), lambda s: (s, 0, 0)),
        ],
        out_shape=[
            jax.ShapeDtypeStruct((SEQ, UNITS, BATCH), jnp.float32),
            jax.ShapeDtypeStruct((UNITS, BATCH), jnp.float32),
            jax.ShapeDtypeStruct((SEQ, EMB, BATCH), jnp.float32),
        ],
        scratch_shapes=[pltpu.VMEM((UNITS, BATCH), jnp.float32)],
    )(emb128, W, U, bt)


def kernel(x, initial, emb_table, W, U, b):
    del initial
    idx = jnp.swapaxes(x, 0, 1).reshape(-1).astype(jnp.int32)
    table128 = _tc_prep(emb_table.T)
    rows = _sc_gather(table128, idx)
    emb128 = rows.reshape(SEQ, BATCH, 2 * EMB)
    seq_t, last_t, emb_t = _tc_gru(emb128, W, U, b.T)
    seq_out = jnp.transpose(seq_t, (2, 0, 1))
    last_state = jnp.transpose(last_t, (1, 0))
    embedded = jnp.transpose(emb_t, (2, 0, 1))
    return (seq_out, last_state, embedded)
